# baseline (speedup 1.0000x reference)
import functools

import jax
import jax.numpy as jnp
from jax import lax
from jax.experimental import pallas as pl
from jax.experimental.pallas import tpu as pltpu
from jax.experimental.pallas import tpu_sc as plsc

NUM_FIXED = 1000000
NUM_TO_LEARN = 1000
EMBED_DIM = 64
PAD_DIM = 128
BATCH = 16384
HIST_LEN = 50
HIST_PAD = 56

NUM_WORKERS = 32
B_PER_WORKER = BATCH // NUM_WORKERS
B_CHUNK = 8
ROWS = B_CHUNK * HIST_LEN
NCHUNKS = B_PER_WORKER // B_CHUNK
LANES = 16


def _embed_kernel(fixed_hbm, train_hbm, idx_hbm, out_hbm,
                  train_v, idx_raw0, idx_raw1, idx_fix0, idx_fix1,
                  gbuf0, gbuf1, gsem0, gsem1, ssem0, ssem1, tsem):
    wid = lax.axis_index("s") * 2 + lax.axis_index("c")
    b_base = wid * B_PER_WORKER
    bufs = ((idx_raw0, idx_fix0, gbuf0, gsem0, ssem0),
            (idx_raw1, idx_fix1, gbuf1, gsem1, ssem1))

    pltpu.async_copy(train_hbm, train_v, tsem).wait()

    def out_slice(k):
        return out_hbm.at[pl.ds(b_base + k * B_CHUNK, B_CHUNK),
                          pl.ds(0, HIST_LEN), pl.ds(0, EMBED_DIM)]

    def gather_copies(idx_fix, gbuf, gsem):
        return [pltpu.make_async_copy(
                    fixed_hbm.at[idx_fix.at[bb]],
                    gbuf.at[bb], gsem)
                for bb in range(B_CHUNK)]

    def prep(k, buf):
        idx_raw, idx_fix, gbuf, gsem, _ = buf
        pltpu.sync_copy(
            idx_hbm.at[pl.ds((b_base + k * B_CHUNK) * HIST_LEN, ROWS)],
            idx_raw)

        def clamp_body(i, gmax):
            flat = i * LANES + lax.iota(jnp.int32, LANES)
            g = idx_raw[pl.ds(i * LANES, LANES)]
            plsc.store_scatter(idx_fix, [flat // HIST_LEN, flat % HIST_LEN],
                               2 * jnp.minimum(g, NUM_FIXED - 1))
            return jnp.maximum(gmax, jnp.max(g))
        gmax = lax.fori_loop(0, ROWS // LANES, clamp_body, 0)
        for c in gather_copies(idx_fix, gbuf, gsem):
            c.start()
        return gmax

    def drain_store(k, buf):
        _, _, gbuf, _, ssem = buf
        pltpu.make_async_copy(gbuf, out_slice(k), ssem).wait()

    def work(k, buf, gmax):
        idx_raw, idx_fix, gbuf, gsem, ssem = buf
        for c in gather_copies(idx_fix, gbuf, gsem):
            c.wait()

        @pl.when(gmax >= NUM_FIXED)
        def _():
            def fix_body(i, _):
                g = idx_raw[pl.ds(i * LANES, LANES)]
                @pl.when(jnp.max(g) >= NUM_FIXED)
                def _():
                    m = g >= NUM_FIXED
                    trow = jnp.maximum(g - NUM_FIXED, 0)
                    flat = i * LANES + lax.iota(jnp.int32, LANES)
                    brow = flat // HIST_LEN
                    hrow = flat % HIST_LEN
                    for c in range(EMBED_DIM):
                        col = jnp.full((LANES,), c, jnp.int32)
                        v = plsc.load_gather(train_v, [trow, col], mask=m)
                        plsc.store_scatter(gbuf, [brow, hrow, col], v,
                                           mask=m)
                return 0
            lax.fori_loop(0, ROWS // LANES, fix_body, 0)

        pltpu.make_async_copy(gbuf, out_slice(k), ssem).start()

    gmax0_init = prep(0, bufs[0])

    def body(i, gmax0):
        k = 2 * i
        @pl.when(i > 0)
        def _():
            drain_store(k - 1, bufs[1])
        gmax1 = prep(k + 1, bufs[1])
        work(k, bufs[0], gmax0)

        def refill():
            drain_store(k, bufs[0])
            return prep(k + 2, bufs[0])
        next_gmax0 = lax.cond(i < NCHUNKS // 2 - 1, refill, lambda: 0)
        work(k + 1, bufs[1], gmax1)
        return next_gmax0

    lax.fori_loop(0, NCHUNKS // 2, body, gmax0_init)
    drain_store(NCHUNKS - 2, bufs[0])
    drain_store(NCHUNKS - 1, bufs[1])


@jax.jit
def kernel(fixed_weights, trainable_weight, inp):
    idx = inp.reshape(BATCH * HIST_LEN).astype(jnp.int32)
    quarter = NUM_FIXED // 4
    fixed_p = jnp.concatenate(
        [jnp.pad(fixed_weights[q * quarter:(q + 1) * quarter],
                 ((0, 0), (0, PAD_DIM - EMBED_DIM)))
         for q in range(4)], axis=0)
    fixed_2x = fixed_p.reshape(2 * NUM_FIXED, EMBED_DIM)
    mesh = plsc.VectorSubcoreMesh(core_axis_name="c", subcore_axis_name="s")
    run = functools.partial(
        pl.kernel, mesh=mesh,
        compiler_params=pltpu.CompilerParams(
            use_tc_tiling_on_sc=False, needs_layout_passes=False),
        out_type=jax.ShapeDtypeStruct((BATCH, HIST_PAD, PAD_DIM),
                                      jnp.float32),
        scratch_types=[
            pltpu.VMEM((NUM_TO_LEARN, EMBED_DIM), jnp.float32),
            pltpu.VMEM((ROWS,), jnp.int32),
            pltpu.VMEM((ROWS,), jnp.int32),
            pltpu.VMEM((B_CHUNK, HIST_LEN), jnp.int32),
            pltpu.VMEM((B_CHUNK, HIST_LEN), jnp.int32),
            pltpu.VMEM((B_CHUNK, HIST_LEN, EMBED_DIM), jnp.float32),
            pltpu.VMEM((B_CHUNK, HIST_LEN, EMBED_DIM), jnp.float32),
            pltpu.SemaphoreType.DMA,
            pltpu.SemaphoreType.DMA,
            pltpu.SemaphoreType.DMA,
            pltpu.SemaphoreType.DMA,
            pltpu.SemaphoreType.DMA,
        ],
    )(_embed_kernel)
    out_p = run(fixed_2x, trainable_weight, idx)
    return out_p[:, :HIST_LEN, :EMBED_DIM]


# speedup vs baseline: 2.8165x; 2.8165x over previous
import functools

import jax
import jax.numpy as jnp
from jax import lax
from jax.experimental import pallas as pl
from jax.experimental.pallas import tpu as pltpu
from jax.experimental.pallas import tpu_sc as plsc

NUM_FIXED = 1000000
NUM_TO_LEARN = 1000
EMBED_DIM = 64
PAD_DIM = 128
BATCH = 16384
HIST_LEN = 50
HIST_PAD = 56

NUM_WORKERS = 32
B_PER_WORKER = BATCH // NUM_WORKERS
B_CHUNK = 8
ROWS = B_CHUNK * HIST_LEN
NCHUNKS = B_PER_WORKER // B_CHUNK
LANES = 16


def _embed_kernel(fixed_hbm, train_hbm, idx_hbm, out_hbm,
                  train_v, idx_raw0, idx_raw1, idx_fix0, idx_fix1,
                  gbuf0, gbuf1, gsem0, gsem1, ssem0, ssem1, tsem):
    wid = lax.axis_index("s") * 2 + lax.axis_index("c")
    b_base = wid * B_PER_WORKER
    bufs = ((idx_raw0, idx_fix0, gbuf0, gsem0, ssem0),
            (idx_raw1, idx_fix1, gbuf1, gsem1, ssem1))

    pltpu.async_copy(train_hbm, train_v, tsem).wait()

    def out_slice(k):
        return out_hbm.at[pl.ds(b_base + k * B_CHUNK, B_CHUNK),
                          pl.ds(0, HIST_LEN), pl.ds(0, EMBED_DIM)]

    def gather_copies(idx_fix, gbuf, gsem):
        return [pltpu.make_async_copy(
                    fixed_hbm.at[idx_fix.at[bb]],
                    gbuf.at[bb], gsem)
                for bb in range(B_CHUNK)]

    def prep(k, buf):
        idx_raw, idx_fix, gbuf, gsem, _ = buf
        pltpu.sync_copy(
            idx_hbm.at[pl.ds((b_base + k * B_CHUNK) * HIST_LEN, ROWS)],
            idx_raw)

        def clamp_body(i, gmax):
            flat = i * LANES + lax.iota(jnp.int32, LANES)
            g = idx_raw[pl.ds(i * LANES, LANES)]
            plsc.store_scatter(idx_fix, [flat // HIST_LEN, flat % HIST_LEN],
                               2 * jnp.minimum(g, NUM_FIXED - 1))
            return jnp.maximum(gmax, jnp.max(g))
        gmax = lax.fori_loop(0, ROWS // LANES, clamp_body, 0)
        for c in gather_copies(idx_fix, gbuf, gsem):
            c.start()
        return gmax

    def drain_store(k, buf):
        _, _, gbuf, _, ssem = buf
        pltpu.make_async_copy(gbuf, out_slice(k), ssem).wait()

    def work(k, buf, gmax):
        idx_raw, idx_fix, gbuf, gsem, ssem = buf
        for c in gather_copies(idx_fix, gbuf, gsem):
            c.wait()

        @pl.when(gmax >= NUM_FIXED)
        def _():
            def fix_body(i, _):
                g = idx_raw[pl.ds(i * LANES, LANES)]
                @pl.when(jnp.max(g) >= NUM_FIXED)
                def _():
                    m = g >= NUM_FIXED
                    trow = jnp.maximum(g - NUM_FIXED, 0)
                    flat = i * LANES + lax.iota(jnp.int32, LANES)
                    brow = flat // HIST_LEN
                    hrow = flat % HIST_LEN
                    for c in range(EMBED_DIM):
                        col = jnp.full((LANES,), c, jnp.int32)
                        v = plsc.load_gather(train_v, [trow, col], mask=m)
                        plsc.store_scatter(gbuf, [brow, hrow, col], v,
                                           mask=m)
                return 0
            lax.fori_loop(0, ROWS // LANES, fix_body, 0)

        pltpu.make_async_copy(gbuf, out_slice(k), ssem).start()

    gmax0_init = prep(0, bufs[0])

    def body(i, gmax0):
        k = 2 * i
        @pl.when(i > 0)
        def _():
            drain_store(k - 1, bufs[1])
        gmax1 = prep(k + 1, bufs[1])
        work(k, bufs[0], gmax0)

        def refill():
            drain_store(k, bufs[0])
            return prep(k + 2, bufs[0])
        next_gmax0 = lax.cond(i < NCHUNKS // 2 - 1, refill, lambda: 0)
        work(k + 1, bufs[1], gmax1)
        return next_gmax0

    lax.fori_loop(0, NCHUNKS // 2, body, gmax0_init)
    drain_store(NCHUNKS - 2, bufs[0])
    drain_store(NCHUNKS - 1, bufs[1])


@jax.jit
def kernel(fixed_weights, trainable_weight, inp):
    idx = inp.reshape(BATCH * HIST_LEN).astype(jnp.int32)
    fixed_p = jnp.pad(fixed_weights, ((0, 0), (0, PAD_DIM - EMBED_DIM)))
    fixed_2x = fixed_p.reshape(2 * NUM_FIXED, EMBED_DIM)
    mesh = plsc.VectorSubcoreMesh(core_axis_name="c", subcore_axis_name="s")
    run = functools.partial(
        pl.kernel, mesh=mesh,
        compiler_params=pltpu.CompilerParams(
            use_tc_tiling_on_sc=False, needs_layout_passes=False),
        out_type=jax.ShapeDtypeStruct((BATCH, HIST_PAD, PAD_DIM),
                                      jnp.float32),
        scratch_types=[
            pltpu.VMEM((NUM_TO_LEARN, EMBED_DIM), jnp.float32),
            pltpu.VMEM((ROWS,), jnp.int32),
            pltpu.VMEM((ROWS,), jnp.int32),
            pltpu.VMEM((B_CHUNK, HIST_LEN), jnp.int32),
            pltpu.VMEM((B_CHUNK, HIST_LEN), jnp.int32),
            pltpu.VMEM((B_CHUNK, HIST_LEN, EMBED_DIM), jnp.float32),
            pltpu.VMEM((B_CHUNK, HIST_LEN, EMBED_DIM), jnp.float32),
            pltpu.SemaphoreType.DMA,
            pltpu.SemaphoreType.DMA,
            pltpu.SemaphoreType.DMA,
            pltpu.SemaphoreType.DMA,
            pltpu.SemaphoreType.DMA,
        ],
    )(_embed_kernel)
    out_p = run(fixed_2x, trainable_weight, idx)
    return out_p[:, :HIST_LEN, :EMBED_DIM]
